# Initial kernel scaffold; baseline (speedup 1.0000x reference)
import jax
import jax.numpy as jnp
from jax.experimental import pallas as pl


def kernel(user_idx, genre_indices, year, user_emb, genre_emb, W1, b1, W2, b2, W3, b3):
    raise NotImplementedError("write your pallas kernel here")


# speedup vs baseline: 5.9082x; 5.9082x over previous
import functools

import jax
import jax.numpy as jnp
from jax import lax
from jax.experimental import pallas as pl
from jax.experimental.pallas import tpu as pltpu
from jax.experimental.pallas import tpu_sc as plsc

_B = 16384
_L = 20
_ED = 128
_HD = 512
_NG = 100

_NC = 2
_NS = 16
_NW = _NC * _NS
_BPW = _B // _NW
_CHUNK = 128
_NCHUNK = _BPW // _CHUNK


def _sc_gather(table, idx3):
    mesh = plsc.VectorSubcoreMesh(core_axis_name="c", subcore_axis_name="s")

    @functools.partial(
        pl.kernel,
        mesh=mesh,
        out_type=jax.ShapeDtypeStruct((_B, _ED), jnp.float32),
        scratch_types=[
            pltpu.VMEM((_NCHUNK, _CHUNK), jnp.int32),
            pltpu.VMEM((_BPW, _ED), jnp.float32),
            pltpu.SemaphoreType.DMA,
        ],
    )
    def k(table_hbm, idx_hbm, out_hbm, idx_v, rows_v, sem):
        wid = lax.axis_index("s") * _NC + lax.axis_index("c")
        base = wid * _BPW
        pltpu.sync_copy(idx_hbm.at[wid], idx_v)
        copies = []
        for j in range(_NCHUNK):
            copies.append(
                pltpu.async_copy(
                    table_hbm.at[idx_v.at[j]],
                    rows_v.at[pl.ds(j * _CHUNK, _CHUNK)],
                    sem,
                )
            )
        for c in copies:
            c.wait()
        pltpu.sync_copy(rows_v, out_hbm.at[pl.ds(base, _BPW)])

    return k(table, idx3)


_BT = 1024


def _mlp_body(user_ref, gidx_ref, year_ref, gemb_ref, w1a_ref, w1b_ref,
              w1c_ref, b1_ref, w2_ref, b2_ref, w3_ref, b3_ref, out_ref):
    g = gidx_ref[...]
    iot = lax.broadcasted_iota(jnp.int32, (1, _NG), 1)
    counts = jnp.zeros((_BT, _NG), jnp.float32)
    for l in range(_L):
        counts += (g[:, l][:, None] == iot).astype(jnp.float32)
    gmean = jnp.dot(counts, gemb_ref[...],
                    preferred_element_type=jnp.float32,
                precision=lax.Precision.HIGHEST) * (1.0 / _L)
    h = jnp.dot(user_ref[...], w1a_ref[...],
                preferred_element_type=jnp.float32,
                precision=lax.Precision.HIGHEST)
    h += jnp.dot(gmean, w1b_ref[...], preferred_element_type=jnp.float32,
                precision=lax.Precision.HIGHEST)
    h += year_ref[...] * w1c_ref[...]
    h += b1_ref[...]
    h = jnp.maximum(h, 0.0)
    h = jnp.dot(h, w2_ref[...], preferred_element_type=jnp.float32,
                precision=lax.Precision.HIGHEST)
    h = jnp.maximum(h + b2_ref[...], 0.0)
    out = jnp.dot(h, w3_ref[...], preferred_element_type=jnp.float32,
                precision=lax.Precision.HIGHEST)
    out_ref[...] = out + b3_ref[...]


def _tc_mlp(user_rows, gidx, year2, genre_emb, w1a, w1b, w1c, b1, w2, b2,
            w3, b3):
    grid = (_B // _BT,)
    bs = pl.BlockSpec
    return pl.pallas_call(
        _mlp_body,
        grid=grid,
        in_specs=[
            bs((_BT, _ED), lambda i: (i, 0)),
            bs((_BT, _L), lambda i: (i, 0)),
            bs((_BT, 1), lambda i: (i, 0)),
            bs((_NG, _ED), lambda i: (0, 0)),
            bs((_ED, _HD), lambda i: (0, 0)),
            bs((_ED, _HD), lambda i: (0, 0)),
            bs((1, _HD), lambda i: (0, 0)),
            bs((1, _HD), lambda i: (0, 0)),
            bs((_HD, _HD // 2), lambda i: (0, 0)),
            bs((1, _HD // 2), lambda i: (0, 0)),
            bs((_HD // 2, 1), lambda i: (0, 0)),
            bs((1, 1), lambda i: (0, 0)),
        ],
        out_specs=bs((_BT, 1), lambda i: (i, 0)),
        out_shape=jax.ShapeDtypeStruct((_B, 1), jnp.float32),
        compiler_params=pltpu.CompilerParams(
            dimension_semantics=("arbitrary",),
        ),
    )(user_rows, gidx, year2, genre_emb, w1a, w1b, w1c, b1, w2, b2, w3, b3)


def kernel(user_idx, genre_indices, year, user_emb, genre_emb, W1, b1, W2,
           b2, W3, b3):
    idx3 = user_idx.astype(jnp.int32).reshape(_NW, _NCHUNK, _CHUNK)
    user_rows = _sc_gather(user_emb, idx3)
    out = _tc_mlp(
        user_rows,
        genre_indices.astype(jnp.int32),
        year.reshape(_B, 1),
        genre_emb,
        W1[:_ED],
        W1[_ED:2 * _ED],
        W1[2 * _ED:],
        b1.reshape(1, _HD),
        W2,
        b2.reshape(1, _HD // 2),
        W3,
        b3.reshape(1, 1),
    )
    return out.reshape(_B)


# speedup vs baseline: 8.6230x; 1.4595x over previous
import functools

import jax
import jax.numpy as jnp
from jax import lax
from jax.experimental import pallas as pl
from jax.experimental.pallas import tpu as pltpu
from jax.experimental.pallas import tpu_sc as plsc

_B = 16384
_L = 20
_ED = 128
_HD = 512
_NG = 100

_NC = 2
_NS = 16
_NW = _NC * _NS
_BPW = _B // _NW
_CHUNK = 128
_NCHUNK = _BPW // _CHUNK


def _sc_gather(table, idx3):
    mesh = plsc.VectorSubcoreMesh(core_axis_name="c", subcore_axis_name="s")

    @functools.partial(
        pl.kernel,
        mesh=mesh,
        out_type=jax.ShapeDtypeStruct((_B, _ED), jnp.float32),
        scratch_types=[
            pltpu.VMEM((_NCHUNK, _CHUNK), jnp.int32),
            pltpu.VMEM((_BPW, _ED), jnp.float32),
            pltpu.SemaphoreType.DMA,
        ],
    )
    def k(table_hbm, idx_hbm, out_hbm, idx_v, rows_v, sem):
        wid = lax.axis_index("s") * _NC + lax.axis_index("c")
        base = wid * _BPW
        pltpu.sync_copy(idx_hbm.at[wid], idx_v)
        copies = []
        for j in range(_NCHUNK):
            copies.append(
                pltpu.async_copy(
                    table_hbm.at[idx_v.at[j]],
                    rows_v.at[pl.ds(j * _CHUNK, _CHUNK)],
                    sem,
                )
            )
        for c in copies:
            c.wait()
        pltpu.sync_copy(rows_v, out_hbm.at[pl.ds(base, _BPW)])

    return k(table, idx3)


_BT = 1024


def _split_hi_lo(x):
    hi = x.astype(jnp.bfloat16)
    lo = (x - hi.astype(jnp.float32)).astype(jnp.bfloat16)
    return hi, lo


def _dot_b3(a_hi, a_lo, b_hi, b_lo):
    d = functools.partial(jnp.dot, preferred_element_type=jnp.float32)
    return d(a_hi, b_hi) + d(a_hi, b_lo) + d(a_lo, b_hi)


def _mlp_body(user_ref, gidx_ref, year_ref, gembh_ref, gembl_ref,
              w1ah_ref, w1al_ref, w1bh_ref, w1bl_ref, w1c_ref, b1_ref,
              w2h_ref, w2l_ref, b2_ref, w3h_ref, w3l_ref, b3_ref, out_ref):
    g = gidx_ref[...]
    iot = lax.broadcasted_iota(jnp.int32, (1, _NG), 1)
    counts = jnp.zeros((_BT, _NG), jnp.float32)
    for l in range(_L):
        counts += (g[:, l][:, None] == iot).astype(jnp.float32)
    cb = counts.astype(jnp.bfloat16)
    d = functools.partial(jnp.dot, preferred_element_type=jnp.float32)
    gmean = (d(cb, gembh_ref[...]) + d(cb, gembl_ref[...])) * (1.0 / _L)
    u_hi, u_lo = _split_hi_lo(user_ref[...])
    g_hi, g_lo = _split_hi_lo(gmean)
    h = _dot_b3(u_hi, u_lo, w1ah_ref[...], w1al_ref[...])
    h += _dot_b3(g_hi, g_lo, w1bh_ref[...], w1bl_ref[...])
    h += year_ref[...] * w1c_ref[...]
    h += b1_ref[...]
    h = jnp.maximum(h, 0.0)
    h_hi, h_lo = _split_hi_lo(h)
    h = _dot_b3(h_hi, h_lo, w2h_ref[...], w2l_ref[...])
    h = jnp.maximum(h + b2_ref[...], 0.0)
    h_hi, h_lo = _split_hi_lo(h)
    out = _dot_b3(h_hi, h_lo, w3h_ref[...], w3l_ref[...])
    out_ref[...] = out + b3_ref[...]


def _tc_mlp(user_rows, gidx, year2, gembh, gembl, w1ah, w1al, w1bh, w1bl,
            w1c, b1, w2h, w2l, b2, w3h, w3l, b3):
    grid = (_B // _BT,)
    bs = pl.BlockSpec

    def _const(shape):
        return bs(shape, lambda i: tuple(0 for _ in shape))

    return pl.pallas_call(
        _mlp_body,
        grid=grid,
        in_specs=[
            bs((_BT, _ED), lambda i: (i, 0)),
            bs((_BT, _L), lambda i: (i, 0)),
            bs((_BT, 1), lambda i: (i, 0)),
            _const((_NG, _ED)),
            _const((_NG, _ED)),
            _const((_ED, _HD)),
            _const((_ED, _HD)),
            _const((_ED, _HD)),
            _const((_ED, _HD)),
            _const((1, _HD)),
            _const((1, _HD)),
            _const((_HD, _HD // 2)),
            _const((_HD, _HD // 2)),
            _const((1, _HD // 2)),
            _const((_HD // 2, 1)),
            _const((_HD // 2, 1)),
            _const((1, 1)),
        ],
        out_specs=bs((_BT, 1), lambda i: (i, 0)),
        out_shape=jax.ShapeDtypeStruct((_B, 1), jnp.float32),
        compiler_params=pltpu.CompilerParams(
            dimension_semantics=("arbitrary",),
        ),
    )(user_rows, gidx, year2, gembh, gembl, w1ah, w1al, w1bh, w1bl, w1c,
      b1, w2h, w2l, b2, w3h, w3l, b3)


def kernel(user_idx, genre_indices, year, user_emb, genre_emb, W1, b1, W2,
           b2, W3, b3):
    idx3 = user_idx.astype(jnp.int32).reshape(_NW, _NCHUNK, _CHUNK)
    user_rows = _sc_gather(user_emb, idx3)
    gembh, gembl = _split_hi_lo(genre_emb)
    w1ah, w1al = _split_hi_lo(W1[:_ED])
    w1bh, w1bl = _split_hi_lo(W1[_ED:2 * _ED])
    w2h, w2l = _split_hi_lo(W2)
    w3h, w3l = _split_hi_lo(W3)
    out = _tc_mlp(
        user_rows,
        genre_indices.astype(jnp.int32),
        year.reshape(_B, 1),
        gembh, gembl,
        w1ah, w1al, w1bh, w1bl,
        W1[2 * _ED:],
        b1.reshape(1, _HD),
        w2h, w2l,
        b2.reshape(1, _HD // 2),
        w3h, w3l,
        b3.reshape(1, 1),
    )
    return out.reshape(_B)


# speedup vs baseline: 12.5826x; 1.4592x over previous
import functools

import jax
import jax.numpy as jnp
from jax import lax
from jax.experimental import pallas as pl
from jax.experimental.pallas import tpu as pltpu
from jax.experimental.pallas import tpu_sc as plsc

_B = 16384
_L = 20
_ED = 128
_HD = 512
_NG = 100

_NC = 2
_NS = 16
_NW = _NC * _NS
_BPW = _B // _NW
_CHUNK = 128
_NCHUNK = _BPW // _CHUNK
_LANES = 16


_NGP = 128
_CHALF = _BPW // 2


def _sc_gather_and_count(table, idx3, gt3):
    mesh = plsc.VectorSubcoreMesh(core_axis_name="c", subcore_axis_name="s")

    @functools.partial(
        pl.kernel,
        mesh=mesh,
        out_type=(
            jax.ShapeDtypeStruct((_B, _ED), jnp.float32),
            jax.ShapeDtypeStruct((_NW, _BPW, _NGP), jnp.float32),
        ),
        scratch_types=[
            pltpu.VMEM((_NCHUNK, _CHUNK), jnp.int32),
            pltpu.VMEM((_BPW, _ED), jnp.float32),
            pltpu.VMEM((_L, _BPW), jnp.int32),
            pltpu.VMEM((_CHALF, _NGP), jnp.float32),
            pltpu.SemaphoreType.DMA,
        ],
        compiler_params=pltpu.CompilerParams(needs_layout_passes=False),
    )
    def k(table_hbm, idx_hbm, gt_hbm, rows_out, cnt_out, idx_v, rows_v,
          gt_v, cnt_v, sem):
        wid = lax.axis_index("s") * _NC + lax.axis_index("c")
        base = wid * _BPW
        pltpu.sync_copy(idx_hbm.at[wid], idx_v)
        copies = []
        for j in range(_NCHUNK):
            copies.append(
                pltpu.async_copy(
                    table_hbm.at[idx_v.at[j]],
                    rows_v.at[pl.ds(j * _CHUNK, _CHUNK)],
                    sem,
                )
            )
        pltpu.sync_copy(gt_hbm.at[wid], gt_v)

        zero = jnp.zeros((_LANES,), jnp.float32)
        ones = jnp.full((_LANES,), 1.0, jnp.float32)
        lane_iota = lax.iota(jnp.int32, _LANES)

        for h in range(2):
            def zbody(i, carry):
                for cc in range(_NGP // _LANES):
                    cnt_v[i, pl.ds(cc * _LANES, _LANES)] = zero
                return carry

            lax.fori_loop(0, _CHALF, zbody, 0)

            def sbody(grp, carry):
                rowvec = grp * _LANES + lane_iota
                for l in range(_L):
                    gv = gt_v[l, pl.ds(h * _CHALF + grp * _LANES, _LANES)]
                    cur = plsc.load_gather(cnt_v, [rowvec, gv])
                    plsc.store_scatter(cnt_v, [rowvec, gv], cur + 1.0)
                return carry

            lax.fori_loop(0, _CHALF // _LANES, sbody, 0)
            pltpu.sync_copy(cnt_v, cnt_out.at[wid, pl.ds(h * _CHALF,
                                                         _CHALF)])

        for c in copies:
            c.wait()
        pltpu.sync_copy(rows_v, rows_out.at[pl.ds(base, _BPW)])

    return k(table, idx3, gt3)


_BT = 1024


def _split_hi_lo(x):
    hi = x.astype(jnp.bfloat16)
    lo = (x - hi.astype(jnp.float32)).astype(jnp.bfloat16)
    return hi, lo


def _dot_b3(a_hi, a_lo, b_hi, b_lo):
    d = functools.partial(jnp.dot, preferred_element_type=jnp.float32)
    return d(a_hi, b_hi) + d(a_hi, b_lo) + d(a_lo, b_hi)


def _mlp_body(user_ref, cnt_ref, year_ref, gembh_ref, gembl_ref,
              w1ah_ref, w1al_ref, w1bh_ref, w1bl_ref, w1c_ref, b1_ref,
              w2h_ref, w2l_ref, b2_ref, w3h_ref, w3l_ref, b3_ref, out_ref):
    cb = cnt_ref[...].astype(jnp.bfloat16)
    d = functools.partial(jnp.dot, preferred_element_type=jnp.float32)
    gmean = (d(cb, gembh_ref[...]) + d(cb, gembl_ref[...])) * (1.0 / _L)
    u_hi, u_lo = _split_hi_lo(user_ref[...])
    g_hi, g_lo = _split_hi_lo(gmean)
    h = _dot_b3(u_hi, u_lo, w1ah_ref[...], w1al_ref[...])
    h += _dot_b3(g_hi, g_lo, w1bh_ref[...], w1bl_ref[...])
    h += year_ref[...] * w1c_ref[...]
    h += b1_ref[...]
    h = jnp.maximum(h, 0.0)
    h_hi, h_lo = _split_hi_lo(h)
    h = _dot_b3(h_hi, h_lo, w2h_ref[...], w2l_ref[...])
    h = jnp.maximum(h + b2_ref[...], 0.0)
    h_hi, h_lo = _split_hi_lo(h)
    out = _dot_b3(h_hi, h_lo, w3h_ref[...], w3l_ref[...])
    out_ref[...] = out + b3_ref[...]


def _tc_mlp(user_rows, counts, year2, gembh, gembl, w1ah, w1al, w1bh, w1bl,
            w1c, b1, w2h, w2l, b2, w3h, w3l, b3):
    grid = (_B // _BT,)
    bs = pl.BlockSpec

    def _const(shape):
        return bs(shape, lambda i: tuple(0 for _ in shape))

    return pl.pallas_call(
        _mlp_body,
        grid=grid,
        in_specs=[
            bs((_BT, _ED), lambda i: (i, 0)),
            bs((_BT, _NGP), lambda i: (i, 0)),
            bs((_BT, 1), lambda i: (i, 0)),
            _const((_NGP, _ED)),
            _const((_NGP, _ED)),
            _const((_ED, _HD)),
            _const((_ED, _HD)),
            _const((_ED, _HD)),
            _const((_ED, _HD)),
            _const((1, _HD)),
            _const((1, _HD)),
            _const((_HD, _HD // 2)),
            _const((_HD, _HD // 2)),
            _const((1, _HD // 2)),
            _const((_HD // 2, 1)),
            _const((_HD // 2, 1)),
            _const((1, 1)),
        ],
        out_specs=bs((_BT, 1), lambda i: (i, 0)),
        out_shape=jax.ShapeDtypeStruct((_B, 1), jnp.float32),
        compiler_params=pltpu.CompilerParams(
            dimension_semantics=("arbitrary",),
        ),
    )(user_rows, counts, year2, gembh, gembl, w1ah, w1al, w1bh, w1bl, w1c,
      b1, w2h, w2l, b2, w3h, w3l, b3)


def kernel(user_idx, genre_indices, year, user_emb, genre_emb, W1, b1, W2,
           b2, W3, b3):
    idx3 = user_idx.astype(jnp.int32).reshape(_NW, _NCHUNK, _CHUNK)
    gt3 = (genre_indices.astype(jnp.int32).T
           .reshape(_L, _NW, _BPW).transpose(1, 0, 2))
    user_rows, counts_sc = _sc_gather_and_count(user_emb, idx3, gt3)
    counts = counts_sc.reshape(_B, _NGP)
    gembp = jnp.concatenate(
        [genre_emb, jnp.zeros((_NGP - _NG, _ED), jnp.float32)], axis=0)
    gembh, gembl = _split_hi_lo(gembp)
    w1ah, w1al = _split_hi_lo(W1[:_ED])
    w1bh, w1bl = _split_hi_lo(W1[_ED:2 * _ED])
    w2h, w2l = _split_hi_lo(W2)
    w3h, w3l = _split_hi_lo(W3)
    out = _tc_mlp(
        user_rows,
        counts,
        year.reshape(_B, 1),
        gembh, gembl,
        w1ah, w1al, w1bh, w1bl,
        W1[2 * _ED:],
        b1.reshape(1, _HD),
        w2h, w2l,
        b2.reshape(1, _HD // 2),
        w3h, w3l,
        b3.reshape(1, 1),
    )
    return out.reshape(_B)
